# baseline (speedup 1.0000x reference)
import functools
import math

import jax
import jax.numpy as jnp
from jax.experimental import pallas as pl
from jax.experimental.pallas import tpu as pltpu

_D_MODEL = 768
_N_EXP = 64
_Z_COEF = 0.001
_AUX_COEF = 0.01
_CAP_FACTOR = 1.0
_MIN_CAP = 4

_TB = 8192
_LANES = 128


def _router_body(x_ref, wt_ref, idx_ref, prob_ref, cnt_ref, aux_ref,
                 cnt_acc, p_acc, z_acc, *, n_tokens):
    i = pl.program_id(0)
    nb = pl.num_programs(0)
    tb = x_ref.shape[0]

    lg = jax.lax.dot_general(
        wt_ref[...], x_ref[...],
        dimension_numbers=(((0,), (1,)), ((), ())),
        preferred_element_type=jnp.float32)
    m = jnp.max(lg, axis=0, keepdims=True)
    e = jnp.exp(lg - m)
    s = jnp.sum(e, axis=0, keepdims=True)
    eid = jax.lax.broadcasted_iota(jnp.int32, lg.shape, 0)
    amax = jnp.min(jnp.where(lg >= m, eid, _N_EXP), axis=0)
    idx_ref[...] = amax
    prob_ref[...] = 1.0 / s[0]
    lse = m[0] + jnp.log(s[0])

    onehot = (eid == amax[None, :]).astype(jnp.int32)
    cnt_blk = jnp.sum(onehot.reshape(_N_EXP, tb // _LANES, _LANES),
                      axis=1)
    p_blk = jnp.sum((e * (1.0 / s)).reshape(_N_EXP, tb // _LANES, _LANES),
                    axis=1)
    z_blk = jnp.sum((lse * lse).reshape(tb // _LANES, _LANES),
                    axis=0, keepdims=True)

    @pl.when(i == 0)
    def _init():
        cnt_acc[...] = jnp.zeros_like(cnt_acc)
        p_acc[...] = jnp.zeros_like(p_acc)
        z_acc[...] = jnp.zeros_like(z_acc)

    cnt_acc[...] += cnt_blk
    p_acc[...] += p_blk
    z_acc[...] += z_blk

    @pl.when(i == nb - 1)
    def _finish():
        counts = jnp.sum(cnt_acc[...], axis=1)
        cnt_ref[...] = counts
        p_vec = jnp.sum(p_acc[...], axis=1)
        z_sum = jnp.sum(z_acc[...])
        inv_n = 1.0 / n_tokens
        aux = (_AUX_COEF * _N_EXP * jnp.sum(counts.astype(jnp.float32) * p_vec)
               * (inv_n * inv_n) + _Z_COEF * z_sum * inv_n)
        aux_ref[...] = jnp.reshape(aux, (1, 1))


def kernel(x, W):
    B, T, D = x.shape
    n = B * T
    x_flat = x.reshape(n, D)
    wt = W.T
    nb = n // _TB

    body = functools.partial(_router_body, n_tokens=float(n))
    idx, prob, counts, aux = pl.pallas_call(
        body,
        grid=(nb,),
        in_specs=[
            pl.BlockSpec((_TB, D), lambda i: (i, 0)),
            pl.BlockSpec((D, _N_EXP), lambda i: (0, 0)),
        ],
        out_specs=[
            pl.BlockSpec((_TB,), lambda i: (i,)),
            pl.BlockSpec((_TB,), lambda i: (i,)),
            pl.BlockSpec((_N_EXP,), lambda i: (0,)),
            pl.BlockSpec((1, 1), lambda i: (0, 0)),
        ],
        out_shape=[
            jax.ShapeDtypeStruct((n,), jnp.int32),
            jax.ShapeDtypeStruct((n,), jnp.float32),
            jax.ShapeDtypeStruct((_N_EXP,), jnp.int32),
            jax.ShapeDtypeStruct((1, 1), jnp.float32),
        ],
        scratch_shapes=[
            pltpu.VMEM((_N_EXP, _LANES), jnp.int32),
            pltpu.VMEM((_N_EXP, _LANES), jnp.float32),
            pltpu.VMEM((1, _LANES), jnp.float32),
        ],
    )(x_flat, wt)

    capacity = max(_MIN_CAP, math.ceil(_CAP_FACTOR * n / _N_EXP))
    return (idx, prob, counts, jnp.array(capacity, dtype=jnp.int32),
            aux[0, 0])


# speedup vs baseline: 1.0701x; 1.0701x over previous
import functools
import math

import jax
import jax.numpy as jnp
from jax.experimental import pallas as pl
from jax.experimental.pallas import tpu as pltpu

_D_MODEL = 768
_N_EXP = 64
_Z_COEF = 0.001
_AUX_COEF = 0.01
_CAP_FACTOR = 1.0
_MIN_CAP = 4

_TB = 4096
_LANES = 128


def _router_body(x_ref, wt_ref, idx_ref, prob_ref, cnt_ref, aux_ref,
                 cnt_acc, p_acc, z_acc, *, n_tokens):
    i = pl.program_id(0)
    nb = pl.num_programs(0)
    tb = x_ref.shape[0]

    lg = jax.lax.dot_general(
        wt_ref[...], x_ref[...],
        dimension_numbers=(((0,), (1,)), ((), ())),
        preferred_element_type=jnp.float32)
    m = jnp.max(lg, axis=0, keepdims=True)
    e = jnp.exp(lg - m)
    s = jnp.sum(e, axis=0, keepdims=True)
    eid = jax.lax.broadcasted_iota(jnp.int32, lg.shape, 0)
    amax = jnp.min(jnp.where(lg >= m, eid, _N_EXP), axis=0)
    idx_ref[...] = amax
    prob_ref[...] = 1.0 / s[0]
    lse = m[0] + jnp.log(s[0])

    onehot = (eid == amax[None, :]).astype(jnp.int32)
    cnt_blk = jnp.sum(onehot.reshape(_N_EXP, tb // _LANES, _LANES),
                      axis=1)
    p_blk = jnp.sum((e * (1.0 / s)).reshape(_N_EXP, tb // _LANES, _LANES),
                    axis=1)
    z_blk = jnp.sum((lse * lse).reshape(tb // _LANES, _LANES),
                    axis=0, keepdims=True)

    @pl.when(i == 0)
    def _init():
        cnt_acc[...] = jnp.zeros_like(cnt_acc)
        p_acc[...] = jnp.zeros_like(p_acc)
        z_acc[...] = jnp.zeros_like(z_acc)

    cnt_acc[...] += cnt_blk
    p_acc[...] += p_blk
    z_acc[...] += z_blk

    @pl.when(i == nb - 1)
    def _finish():
        counts = jnp.sum(cnt_acc[...], axis=1)
        cnt_ref[...] = counts
        p_vec = jnp.sum(p_acc[...], axis=1)
        z_sum = jnp.sum(z_acc[...])
        inv_n = 1.0 / n_tokens
        aux = (_AUX_COEF * _N_EXP * jnp.sum(counts.astype(jnp.float32) * p_vec)
               * (inv_n * inv_n) + _Z_COEF * z_sum * inv_n)
        aux_ref[...] = jnp.reshape(aux, (1, 1))


def kernel(x, W):
    B, T, D = x.shape
    n = B * T
    x_flat = x.reshape(n, D)
    wt = W.T
    nb = n // _TB

    body = functools.partial(_router_body, n_tokens=float(n))
    idx, prob, counts, aux = pl.pallas_call(
        body,
        grid=(nb,),
        in_specs=[
            pl.BlockSpec((_TB, D), lambda i: (i, 0)),
            pl.BlockSpec((D, _N_EXP), lambda i: (0, 0)),
        ],
        out_specs=[
            pl.BlockSpec((_TB,), lambda i: (i,)),
            pl.BlockSpec((_TB,), lambda i: (i,)),
            pl.BlockSpec((_N_EXP,), lambda i: (0,)),
            pl.BlockSpec((1, 1), lambda i: (0, 0)),
        ],
        out_shape=[
            jax.ShapeDtypeStruct((n,), jnp.int32),
            jax.ShapeDtypeStruct((n,), jnp.float32),
            jax.ShapeDtypeStruct((_N_EXP,), jnp.int32),
            jax.ShapeDtypeStruct((1, 1), jnp.float32),
        ],
        scratch_shapes=[
            pltpu.VMEM((_N_EXP, _LANES), jnp.int32),
            pltpu.VMEM((_N_EXP, _LANES), jnp.float32),
            pltpu.VMEM((1, _LANES), jnp.float32),
        ],
    )(x_flat, wt)

    capacity = max(_MIN_CAP, math.ceil(_CAP_FACTOR * n / _N_EXP))
    return (idx, prob, counts, jnp.array(capacity, dtype=jnp.int32),
            aux[0, 0])
